# baseline (speedup 1.0000x reference)
import functools

import jax
import jax.numpy as jnp
from jax import lax
from jax.experimental import pallas as pl
from jax.experimental.pallas import tpu as pltpu
from jax.experimental.pallas import tpu_sc as plsc

T = 1024
D = 768
H = 3072
G = 4
EPG = 4
E = G * EPG
KG = 2
KE = 2
K = KG * KE
CAP = 256

NW = 32
SLOTS = E * CAP
SPW = SLOTS // NW
TPW = T // NW


def _top2_lanes(v, width):
    lane = jax.lax.broadcasted_iota(jnp.int32, (T, width), 1)
    v1 = jnp.max(v, axis=1, keepdims=True)
    i1 = jnp.min(jnp.where(v == v1, lane, width), axis=1, keepdims=True)
    v_masked = jnp.where(lane == i1, -jnp.inf, v)
    v2 = jnp.max(v_masked, axis=1, keepdims=True)
    i2 = jnp.min(jnp.where(v_masked == v2, lane, width), axis=1, keepdims=True)
    return (v1, i1), (v2, i2)


def _routing_kernel(x_ref, wgg_ref, wge_ref, tok_ref, wslot_ref, slot_ref):
    x = x_ref[:]
    gl = jnp.dot(x, wgg_ref[:], preferred_element_type=jnp.float32)
    gp = jax.nn.softmax(gl, axis=-1)
    (gv1, gi1), (gv2, gi2) = _top2_lanes(gp, G)

    el = jnp.dot(x, wge_ref[:], preferred_element_type=jnp.float32)
    ep = [jax.nn.softmax(el[:, g * EPG:(g + 1) * EPG], axis=-1) for g in range(G)]

    ws, idxs = [], []
    for gi, gv in ((gi1, gv1), (gi2, gv2)):
        sel = jnp.zeros((T, EPG), jnp.float32)
        for g in range(G):
            sel = jnp.where(gi == g, ep[g], sel)
        (ev1, ei1), (ev2, ei2) = _top2_lanes(sel, EPG)
        for ev, ei in ((ev1, ei1), (ev2, ei2)):
            ws.append(gv * ev)
            idxs.append(gi * EPG + ei)

    denom = ws[0] + ws[1] + ws[2] + ws[3] + 1e-9
    wn = [w / denom for w in ws]
    lane_e = jax.lax.broadcasted_iota(jnp.int32, (T, E), 1)
    route = jnp.zeros((T, E), jnp.float32)
    for w, fi in zip(wn, idxs):
        route = route + jnp.where(lane_e == fi, w, 0.0)

    mask = (route > 0.0).astype(jnp.float32)
    ri = jax.lax.broadcasted_iota(jnp.int32, (T, T), 0)
    ci = jax.lax.broadcasted_iota(jnp.int32, (T, T), 1)
    ltri = (ri >= ci).astype(jnp.float32)
    pos = jnp.dot(ltri, mask, preferred_element_type=jnp.float32) - 1.0

    counts = jnp.sum(mask, axis=0, keepdims=True)
    lane_e1 = jax.lax.broadcasted_iota(jnp.int32, (1, E), 1)
    e0 = jnp.min(jnp.where(counts < CAP, lane_e1, E), axis=1, keepdims=True)
    zslot = jnp.where(e0 < E, e0 * CAP + (CAP - 1), 0)

    lane_k = jax.lax.broadcasted_iota(jnp.int32, (T, K), 1)
    slot_out = jnp.zeros((T, K), jnp.int32)
    for q, (w, fi) in enumerate(zip(wn, idxs)):
        pos_q = jnp.sum(jnp.where(lane_e == fi, pos, 0.0), axis=1, keepdims=True)
        kept = (w > 0.0) & (pos_q < CAP)
        slot_q = jnp.where(kept, fi * CAP + pos_q.astype(jnp.int32), zslot)
        slot_out = slot_out + jnp.where(lane_k == q, slot_q, 0)
    slot_ref[:] = slot_out

    c_iota = jax.lax.broadcasted_iota(jnp.int32, (T, CAP), 1)
    t_iota = jax.lax.broadcasted_iota(jnp.int32, (T, CAP), 0).astype(jnp.float32)
    sub_e = jax.lax.broadcasted_iota(jnp.int32, (E, CAP), 0)
    tok_ec = jnp.zeros((E, CAP), jnp.float32)
    w_ec = jnp.zeros((E, CAP), jnp.float32)
    for e in range(E):
        sel = (lane_e == e)
        r_e = jnp.sum(jnp.where(sel, route, 0.0), axis=1, keepdims=True)
        p_e = jnp.sum(jnp.where(sel, pos, 0.0), axis=1, keepdims=True)
        keep_e = (r_e > 0.0) & (p_e < CAP)
        pt = (p_e.astype(jnp.int32) == c_iota) & keep_e
        tok_row = jnp.sum(jnp.where(pt, t_iota, 0.0), axis=0, keepdims=True)
        w_row = jnp.sum(jnp.where(pt, r_e, 0.0), axis=0, keepdims=True)
        tok_ec = tok_ec + jnp.where(sub_e == e, tok_row, 0.0)
        w_ec = w_ec + jnp.where(sub_e == e, w_row, 0.0)
    tok_ref[:] = tok_ec.astype(jnp.int32)
    wslot_ref[:] = w_ec


def _ffn_kernel(xin_ref, W1_ref, b1_ref, W2_ref, b2_ref, wslot_ref, eo_ref):
    wcol = jnp.transpose(wslot_ref[0])
    h = jax.nn.gelu(
        jnp.dot(xin_ref[:], W1_ref[:], preferred_element_type=jnp.float32)
        + b1_ref[0])
    eo = (jnp.dot(h, W2_ref[:], preferred_element_type=jnp.float32)
          + b2_ref[0])
    eo_ref[:] = wcol * eo


_sc_mesh = plsc.VectorSubcoreMesh(core_axis_name="c", subcore_axis_name="s")


@functools.partial(
    pl.kernel,
    mesh=_sc_mesh,
    out_type=jax.ShapeDtypeStruct((SLOTS, D), jnp.float32),
    scratch_types=[
        pltpu.VMEM((SPW,), jnp.int32),
        pltpu.VMEM((SPW, D), jnp.float32),
        pltpu.SemaphoreType.DMA,
    ],
)
def _sc_gather(x_hbm, tok_hbm, xin_hbm, idx_v, rows_v, sem):
    wid = lax.axis_index("s") * 2 + lax.axis_index("c")
    base = wid * SPW
    pltpu.sync_copy(tok_hbm.at[pl.ds(base, SPW)], idx_v)
    pltpu.async_copy(x_hbm.at[idx_v], rows_v, sem).wait()
    pltpu.sync_copy(rows_v, xin_hbm.at[pl.ds(base, SPW)])


@functools.partial(
    pl.kernel,
    mesh=_sc_mesh,
    out_type=jax.ShapeDtypeStruct((T, D), jnp.float32),
    scratch_types=[
        pltpu.VMEM((TPW * K,), jnp.int32),
        pltpu.VMEM((TPW * K, D), jnp.float32),
        pltpu.VMEM((TPW, D), jnp.float32),
        pltpu.SemaphoreType.DMA,
    ],
)
def _sc_combine(eo_hbm, slot_hbm, out_hbm, idx_v, rows_v, acc_v, sem):
    wid = lax.axis_index("s") * 2 + lax.axis_index("c")
    base = wid * (TPW * K)
    pltpu.sync_copy(slot_hbm.at[pl.ds(base, TPW * K)], idx_v)
    pltpu.async_copy(eo_hbm.at[idx_v], rows_v, sem).wait()

    def body(t, _):
        for c in range(D // 16):
            dsl = pl.ds(c * 16, 16)
            acc_v[t, dsl] = (rows_v[t * K + 0, dsl] + rows_v[t * K + 1, dsl]
                             + rows_v[t * K + 2, dsl] + rows_v[t * K + 3, dsl])
        return 0

    lax.fori_loop(0, TPW, body, 0)
    pltpu.sync_copy(acc_v, out_hbm.at[pl.ds(wid * TPW, TPW)])


def kernel(x, wg_group, wg_expert, W1, b1, W2, b2):
    wge_flat = jnp.transpose(wg_expert, (1, 0, 2)).reshape(D, G * EPG)

    tok, wslot, slot = pl.pallas_call(
        _routing_kernel,
        out_shape=[jax.ShapeDtypeStruct((E, CAP), jnp.int32),
                   jax.ShapeDtypeStruct((E, CAP), jnp.float32),
                   jax.ShapeDtypeStruct((T, K), jnp.int32)],
    )(x, wg_group, wge_flat)

    xin = _sc_gather(x, tok.reshape(SLOTS))

    eo = pl.pallas_call(
        _ffn_kernel,
        grid=(E,),
        in_specs=[
            pl.BlockSpec((CAP, D), lambda e: (e, 0)),
            pl.BlockSpec((D, H), lambda e: (e, 0)),
            pl.BlockSpec((1, 1, H), lambda e: (e, 0, 0)),
            pl.BlockSpec((H, D), lambda e: (e, 0)),
            pl.BlockSpec((1, 1, D), lambda e: (e, 0, 0)),
            pl.BlockSpec((1, 1, CAP), lambda e: (e, 0, 0)),
        ],
        out_specs=pl.BlockSpec((CAP, D), lambda e: (e, 0)),
        out_shape=jax.ShapeDtypeStruct((SLOTS, D), jnp.float32),
        compiler_params=pltpu.CompilerParams(
            dimension_semantics=("arbitrary",)),
    )(xin, W1.reshape(E * D, H), b1.reshape(E, 1, H),
      W2.reshape(E * H, D), b2.reshape(E, 1, D), wslot.reshape(E, 1, CAP))

    out = _sc_combine(eo, slot.reshape(T * K))
    return out


# speedup vs baseline: 1.1373x; 1.1373x over previous
import functools

import jax
import jax.numpy as jnp
from jax import lax
from jax.experimental import pallas as pl
from jax.experimental.pallas import tpu as pltpu
from jax.experimental.pallas import tpu_sc as plsc

T = 1024
D = 768
H = 3072
G = 4
EPG = 4
E = G * EPG
KG = 2
KE = 2
K = KG * KE
CAP = 256

NW = 32
SLOTS = E * CAP
SPW = SLOTS // NW


def _top2_lanes(v, width):
    lane = jax.lax.broadcasted_iota(jnp.int32, (T, width), 1)
    v1 = jnp.max(v, axis=1, keepdims=True)
    i1 = jnp.min(jnp.where(v == v1, lane, width), axis=1, keepdims=True)
    v_masked = jnp.where(lane == i1, -jnp.inf, v)
    v2 = jnp.max(v_masked, axis=1, keepdims=True)
    i2 = jnp.min(jnp.where(v_masked == v2, lane, width), axis=1, keepdims=True)
    return (v1, i1), (v2, i2)


def _routing_kernel(x_ref, wgg_ref, wge_ref, tok_ref, route_ref, pos_ref):
    x = x_ref[:]
    gl = jnp.dot(x, wgg_ref[:], preferred_element_type=jnp.float32)
    gp = jax.nn.softmax(gl, axis=-1)
    (gv1, gi1), (gv2, gi2) = _top2_lanes(gp, G)

    el = jnp.dot(x, wge_ref[:], preferred_element_type=jnp.float32)
    ep = [jax.nn.softmax(el[:, g * EPG:(g + 1) * EPG], axis=-1) for g in range(G)]

    ws, idxs = [], []
    for gi, gv in ((gi1, gv1), (gi2, gv2)):
        sel = jnp.zeros((T, EPG), jnp.float32)
        for g in range(G):
            sel = jnp.where(gi == g, ep[g], sel)
        (ev1, ei1), (ev2, ei2) = _top2_lanes(sel, EPG)
        for ev, ei in ((ev1, ei1), (ev2, ei2)):
            ws.append(gv * ev)
            idxs.append(gi * EPG + ei)

    denom = ws[0] + ws[1] + ws[2] + ws[3] + 1e-9
    lane_e = jax.lax.broadcasted_iota(jnp.int32, (T, E), 1)
    route = jnp.zeros((T, E), jnp.float32)
    for w, fi in zip(ws, idxs):
        route = route + jnp.where(lane_e == fi, w / denom, 0.0)

    mask = (route > 0.0).astype(jnp.float32)
    ri = jax.lax.broadcasted_iota(jnp.int32, (T, T), 0)
    ci = jax.lax.broadcasted_iota(jnp.int32, (T, T), 1)
    ltri = (ri >= ci).astype(jnp.float32)
    pos = jnp.dot(ltri, mask, preferred_element_type=jnp.float32) - 1.0

    c_iota = jax.lax.broadcasted_iota(jnp.int32, (T, CAP), 1)
    t_iota = jax.lax.broadcasted_iota(jnp.int32, (T, CAP), 0).astype(jnp.float32)
    sub_e = jax.lax.broadcasted_iota(jnp.int32, (E, CAP), 0)
    tok_ec = jnp.zeros((E, CAP), jnp.float32)
    for e in range(E):
        sel = (lane_e == e)
        r_e = jnp.sum(jnp.where(sel, route, 0.0), axis=1, keepdims=True)
        p_e = jnp.sum(jnp.where(sel, pos, 0.0), axis=1, keepdims=True)
        keep_e = (r_e > 0.0) & (p_e < CAP)
        pt = (p_e.astype(jnp.int32) == c_iota) & keep_e
        tok_row = jnp.sum(jnp.where(pt, t_iota, 0.0), axis=0, keepdims=True)
        tok_ec = tok_ec + jnp.where(sub_e == e, tok_row, 0.0)
    tok_ref[:] = tok_ec.astype(jnp.int32)
    route_ref[:] = route
    pos_ref[:] = pos


_sc_mesh = plsc.VectorSubcoreMesh(core_axis_name="c", subcore_axis_name="s")


@functools.partial(
    pl.kernel,
    mesh=_sc_mesh,
    out_type=jax.ShapeDtypeStruct((SLOTS, D), jnp.float32),
    scratch_types=[
        pltpu.VMEM((SPW,), jnp.int32),
        pltpu.VMEM((SPW, D), jnp.float32),
        pltpu.SemaphoreType.DMA,
    ],
)
def _sc_gather(x_hbm, tok_hbm, xin_hbm, idx_v, rows_v, sem):
    wid = lax.axis_index("s") * 2 + lax.axis_index("c")
    base = wid * SPW
    pltpu.sync_copy(tok_hbm.at[pl.ds(base, SPW)], idx_v)
    pltpu.async_copy(x_hbm.at[idx_v], rows_v, sem).wait()
    pltpu.sync_copy(rows_v, xin_hbm.at[pl.ds(base, SPW)])


def _ffn_kernel(route_ref, pos_ref, xin_ref, W1_ref, b1_ref, W2_ref, b2_ref,
                out_ref):
    e = pl.program_id(0)

    lane_e = jax.lax.broadcasted_iota(jnp.int32, (T, E), 1)
    sel = (lane_e == e)
    r = jnp.sum(jnp.where(sel, route_ref[:], 0.0), axis=1, keepdims=True)
    p = jnp.sum(jnp.where(sel, pos_ref[:], 0.0), axis=1, keepdims=True)
    c_iota = jax.lax.broadcasted_iota(jnp.int32, (T, CAP), 1)
    keep = (r > 0.0) & (p < CAP)
    ct = jnp.where((p.astype(jnp.int32) == c_iota) & keep, r, 0.0)

    h = jax.nn.gelu(
        jnp.dot(xin_ref[:], W1_ref[:], preferred_element_type=jnp.float32)
        + b1_ref[0])
    eo = (jnp.dot(h, W2_ref[:], preferred_element_type=jnp.float32)
          + b2_ref[0])
    contrib = jnp.dot(ct, eo, preferred_element_type=jnp.float32)

    @pl.when(e == 0)
    def _init():
        out_ref[:] = contrib

    @pl.when(e > 0)
    def _acc():
        out_ref[:] += contrib


def kernel(x, wg_group, wg_expert, W1, b1, W2, b2):
    wge_flat = jnp.transpose(wg_expert, (1, 0, 2)).reshape(D, G * EPG)

    tok, route, pos = pl.pallas_call(
        _routing_kernel,
        out_shape=[jax.ShapeDtypeStruct((E, CAP), jnp.int32),
                   jax.ShapeDtypeStruct((T, E), jnp.float32),
                   jax.ShapeDtypeStruct((T, E), jnp.float32)],
    )(x, wg_group, wge_flat)

    xin = _sc_gather(x, tok.reshape(SLOTS))

    out = pl.pallas_call(
        _ffn_kernel,
        grid=(E,),
        in_specs=[
            pl.BlockSpec((T, E), lambda e: (0, 0)),
            pl.BlockSpec((T, E), lambda e: (0, 0)),
            pl.BlockSpec((CAP, D), lambda e: (e, 0)),
            pl.BlockSpec((D, H), lambda e: (e, 0)),
            pl.BlockSpec((1, 1, H), lambda e: (e, 0, 0)),
            pl.BlockSpec((H, D), lambda e: (e, 0)),
            pl.BlockSpec((1, 1, D), lambda e: (e, 0, 0)),
        ],
        out_specs=pl.BlockSpec((T, D), lambda e: (0, 0)),
        out_shape=jax.ShapeDtypeStruct((T, D), jnp.float32),
        compiler_params=pltpu.CompilerParams(
            dimension_semantics=("arbitrary",)),
    )(route, pos, xin, W1.reshape(E * D, H), b1.reshape(E, 1, H),
      W2.reshape(E * H, D), b2.reshape(E, 1, D))
    return out
